# baseline (speedup 1.0000x reference)
import jax
import jax.numpy as jnp
from jax import lax
from jax.experimental import pallas as pl

Q = 1024
D = 128
K = 100000
KT = 2048
NTILES = 49
KPAD = NTILES * KT
G1 = 64
G2 = 1024
NB1 = KPAD // G1
NB2 = KPAD // G2


def _phase1_body(q_ref, k_ref, s_ref, m1_ref, m2_ref):
    j = pl.program_id(0)
    q = q_ref[...]
    kk = k_ref[...]
    dot = lax.dot_general(
        q, kk, (((1,), (1,)), ((), ())),
        preferred_element_type=jnp.float32,
        precision=lax.Precision.DEFAULT,
    )
    qsq = jnp.sum(q * q, axis=1, keepdims=True)
    ksq = jnp.sum(kk * kk, axis=1)[None, :]
    s = 2.0 * dot - ksq - qsq
    col = j * KT + lax.broadcasted_iota(jnp.int32, (Q, KT), 1)
    s = jnp.where(col < K, s, -3.0e38)
    s_ref[...] = s
    m1 = jnp.max(s.reshape(Q, KT // G1, G1), axis=-1)
    m1_ref[0] = m1
    m2_ref[0] = jnp.max(m1.reshape(Q, KT // G2, G2 // G1), axis=-1)


def _phase1(queries, keys_padded):
    return pl.pallas_call(
        _phase1_body,
        grid=(NTILES,),
        in_specs=[
            pl.BlockSpec((Q, D), lambda j: (0, 0)),
            pl.BlockSpec((KT, D), lambda j: (j, 0)),
        ],
        out_specs=[
            pl.BlockSpec((Q, KT), lambda j: (0, j)),
            pl.BlockSpec((1, Q, KT // G1), lambda j: (j, 0, 0)),
            pl.BlockSpec((1, Q, KT // G2), lambda j: (j, 0, 0)),
        ],
        out_shape=[
            jax.ShapeDtypeStruct((Q, KPAD), jnp.float32),
            jax.ShapeDtypeStruct((NTILES, Q, KT // G1), jnp.float32),
            jax.ShapeDtypeStruct((NTILES, Q, KT // G2), jnp.float32),
        ],
    )(queries, keys_padded)


def kernel(queries, keys, k):
    keys_padded = jnp.pad(keys, ((0, KPAD - K), (0, 0)))
    scores, m1, m2 = _phase1(queries, keys_padded)
    del m1, m2
    vals, idx = jax.lax.top_k(scores, 16)
    idx = idx + jnp.asarray(k - k, idx.dtype)
    return vals, idx


# speedup vs baseline: 11.2124x; 11.2124x over previous
import jax
import jax.numpy as jnp
from jax import lax
from jax.experimental import pallas as pl
from jax.experimental.pallas import tpu as pltpu
from jax.experimental.pallas import tpu_sc as plsc

Q = 1024
D = 128
K = 100000
KT = 2048
NTILES = 49
KPAD = NTILES * KT
G1 = 128
G2 = 1024
NB1 = KPAD // G1
NB2 = KPAD // G2
NB2P = 112
BT = KT // G1
NEG = -3.0e38

NC = 2
NS = 16
L = 16
NW = NC * NS
QW = Q // NW


def _phase1_body(q_ref, k_ref, s_ref, m1_ref, m2_ref):
    j = pl.program_id(0)
    q = q_ref[...]
    qsq = jnp.sum(q * q, axis=1, keepdims=True)
    cols = []
    for sub in range(KT // 256):
        kk = k_ref[pl.ds(sub * 256, 256), :]
        dot = lax.dot_general(
            q, kk, (((1,), (1,)), ((), ())),
            preferred_element_type=jnp.float32,
            precision=lax.Precision.DEFAULT,
        )
        ksq = jnp.sum(kk * kk, axis=1)[None, :]
        s = 2.0 * dot - ksq - qsq
        for h in range(2):
            lb = sub * 2 + h
            sh = s[:, h * G1:(h + 1) * G1]
            base = j * KT + lb * G1
            col = base + lax.broadcasted_iota(jnp.int32, (Q, G1), 1)
            sh = jnp.where(col < K, sh, NEG)
            s_ref[lb] = sh
            cols.append(jnp.max(sh, axis=1, keepdims=True))
    m1 = jnp.concatenate(cols, axis=1)
    m1_ref[0] = m1
    m2_ref[0] = jnp.max(m1.reshape(Q, KT // G2, G2 // G1), axis=-1)


def _phase1(queries, keys_padded):
    return pl.pallas_call(
        _phase1_body,
        grid=(NTILES,),
        in_specs=[
            pl.BlockSpec((Q, D), lambda j: (0, 0)),
            pl.BlockSpec((KT, D), lambda j: (j, 0)),
        ],
        out_specs=[
            pl.BlockSpec((BT, Q, G1), lambda j: (j, 0, 0)),
            pl.BlockSpec((1, Q, BT), lambda j: (j, 0, 0)),
            pl.BlockSpec((1, Q, KT // G2), lambda j: (j, 0, 0)),
        ],
        out_shape=[
            jax.ShapeDtypeStruct((NB1, Q, G1), jnp.float32),
            jax.ShapeDtypeStruct((NTILES, Q, BT), jnp.float32),
            jax.ShapeDtypeStruct((NTILES, Q, KT // G2), jnp.float32),
        ],
    )(queries, keys_padded)


_GATHER_DNUMS = lax.GatherDimensionNumbers(
    offset_dims=(), collapsed_slice_dims=(0,), start_index_map=(0,))


def _permute(x, perm):
    return lax.gather(x, perm[:, None], _GATHER_DNUMS, slice_sizes=(1,),
                      mode=lax.GatherScatterMode.PROMISE_IN_BOUNDS)


def _merge16(av, ai, bv, bi):
    rbv = lax.rev(bv, (0,))
    rbi = lax.rev(bi, (0,))
    take = (av > rbv) | ((av == rbv) & (ai < rbi))
    cv = jnp.where(take, av, rbv)
    ci = jnp.where(take, ai, rbi)
    cv, ci = plsc.sort_key_val(cv, ci, descending=True)
    return cv, ci


def _merge_tree(pairs):
    while len(pairs) > 1:
        nxt = []
        for a in range(0, len(pairs) - 1, 2):
            (av, ai), (bv, bi) = pairs[a], pairs[a + 1]
            nxt.append(_merge16(av, ai, bv, bi))
        if len(pairs) % 2:
            nxt.append(pairs[-1])
        pairs = nxt
    return pairs[0]


def _tie_fix(v, i):
    lane = lax.iota(jnp.int32, L)
    for parity in (0, 1, 0, 1):
        partner = lane - 2 * ((lane - parity) % 2) + 1
        partner = jnp.clip(partner, 0, L - 1)
        pv = _permute(v, partner)
        pi = _permute(i, partner)
        win = (v > pv) | ((v == pv) & (i < pi)) | (partner == lane)
        first = partner > lane
        keep_self = jnp.where(first, win, ~win)
        v = jnp.where(keep_self, v, pv)
        i = jnp.where(keep_self, i, pi)
    return v, i


def _topk_sc_body(m2_hbm, m1_hbm, sc_hbm, vals_hbm, idx_hbm,
                  m2_v, m1_v, gidx_v, gath_v, ov_v, oi_v, sem):
    wid = lax.axis_index("c") * NS + lax.axis_index("s")
    base = wid * QW
    pltpu.sync_copy(m2_hbm.at[pl.ds(base * NB2P, QW * NB2P)], m2_v)
    pltpu.sync_copy(m1_hbm.at[pl.ds(base * NB1, QW * NB1)], m1_v)

    lane = lax.iota(jnp.int32, L)

    def one_query(r, _):
        q = base + r
        pairs = []
        for c in range(NB2P // L):
            v = m2_v[pl.ds(r * NB2P + c * L, L)]
            i = lane + c * L
            pairs.append(plsc.sort_key_val(v, i, descending=True))
        sbv, sbi = _merge_tree(pairs)

        pairs = []
        for g in range(G2 // G1):
            cols = r * NB1 + sbi * (G2 // G1) + g
            v = plsc.load_gather(m1_v, [cols])
            i = sbi * (G2 // G1) + g
            pairs.append(plsc.sort_key_val(v, i, descending=True))
        bv, bi = _merge_tree(pairs)

        gidx_v[...] = bi * Q + q
        pltpu.async_copy(sc_hbm.at[gidx_v], gath_v, sem).wait()

        pairs = []
        for blk in range(L):
            b = _permute(bi, jnp.full((L,), blk, jnp.int32))
            for seg in range(G1 // L):
                v = gath_v[blk, pl.ds(seg * L, L)]
                i = b * G1 + (seg * L) + lane
                pairs.append(plsc.sort_key_val(v, i, descending=True))
        fv, fi = _merge_tree(pairs)
        fv, fi = _tie_fix(fv, fi)

        ov_v[pl.ds(r * L, L)] = fv
        oi_v[pl.ds(r * L, L)] = fi
        return 0

    lax.fori_loop(0, QW, one_query, 0)
    pltpu.sync_copy(ov_v, vals_hbm.at[pl.ds(base * L, QW * L)])
    pltpu.sync_copy(oi_v, idx_hbm.at[pl.ds(base * L, QW * L)])


def _topk_sc(m2_flat, m1_flat, scores_rows):
    kern = pl.kernel(
        _topk_sc_body,
        out_type=[
            jax.ShapeDtypeStruct((Q * L,), jnp.float32),
            jax.ShapeDtypeStruct((Q * L,), jnp.int32),
        ],
        mesh=plsc.VectorSubcoreMesh(core_axis_name="c", subcore_axis_name="s"),
        compiler_params=pltpu.CompilerParams(needs_layout_passes=False),
        scratch_types=[
            pltpu.VMEM((QW * NB2P,), jnp.float32),
            pltpu.VMEM((QW * NB1,), jnp.float32),
            pltpu.VMEM((L,), jnp.int32),
            pltpu.VMEM((L, G1), jnp.float32),
            pltpu.VMEM((QW * L,), jnp.float32),
            pltpu.VMEM((QW * L,), jnp.int32),
            pltpu.SemaphoreType.DMA,
        ],
    )
    return kern(m2_flat, m1_flat, scores_rows)


def kernel(queries, keys, k):
    keys_padded = jnp.pad(keys, ((0, KPAD - K), (0, 0)))
    scores, m1_3d, m2_3d = _phase1(queries, keys_padded)
    m1 = m1_3d.transpose(1, 0, 2).reshape(Q, NB1)
    m2 = m2_3d.transpose(1, 0, 2).reshape(Q, NB2)
    m2 = jnp.pad(m2, ((0, 0), (0, NB2P - NB2)), constant_values=NEG)
    vals_flat, idx_flat = _topk_sc(
        m2.reshape(-1), m1.reshape(-1), scores.reshape(NB1 * Q, G1))
    vals = vals_flat.reshape(Q, L)
    idx = idx_flat.reshape(Q, L)
    idx = idx + jnp.asarray(k - k, idx.dtype)
    return vals, idx


# speedup vs baseline: 14.2060x; 1.2670x over previous
import jax
import jax.numpy as jnp
from jax import lax
from jax.experimental import pallas as pl
from jax.experimental.pallas import tpu as pltpu
from jax.experimental.pallas import tpu_sc as plsc

Q = 1024
D = 128
K = 100000
KT = 2048
NTILES = 49
KPAD = NTILES * KT
G1 = 128
G2 = 1024
NB1 = KPAD // G1
NB2 = KPAD // G2
NB2P = 112
BT = KT // G1
NEG = -3.0e38

NC = 2
NS = 16
L = 16
NW = NC * NS
QW = Q // NW


def _phase1_body(q_ref, k_ref, s_ref, m1_ref, m2_ref):
    q2 = q_ref[...] * 2.0
    rows = []
    for sub in range(KT // 256):
        kk = k_ref[pl.ds(sub * 256, 256), :]
        dot2 = lax.dot_general(
            q2, kk, (((1,), (1,)), ((), ())),
            preferred_element_type=jnp.float32,
            precision=lax.Precision.DEFAULT,
        )
        ksq = jnp.sum(kk * kk, axis=1)[None, :]
        s = dot2 - ksq
        for h in range(2):
            lb = sub * 2 + h
            sh = s[:, h * G1:(h + 1) * G1]
            s_ref[lb] = sh
            rows.append(jnp.max(sh, axis=1, keepdims=True))
    m1_ref[0] = jnp.concatenate(rows, axis=1)
    m2a = rows[0]
    m2b = rows[BT // 2]
    for lb in range(1, BT // 2):
        m2a = jnp.maximum(m2a, rows[lb])
        m2b = jnp.maximum(m2b, rows[BT // 2 + lb])
    m2_ref[0] = jnp.concatenate([m2a, m2b], axis=1)


def _phase1(queries, keys_padded):
    return pl.pallas_call(
        _phase1_body,
        grid=(NTILES,),
        in_specs=[
            pl.BlockSpec((Q, D), lambda j: (0, 0)),
            pl.BlockSpec((KT, D), lambda j: (j, 0)),
        ],
        out_specs=[
            pl.BlockSpec((BT, Q, G1), lambda j: (j, 0, 0)),
            pl.BlockSpec((1, Q, BT), lambda j: (j, 0, 0)),
            pl.BlockSpec((1, Q, KT // G2), lambda j: (j, 0, 0)),
        ],
        out_shape=[
            jax.ShapeDtypeStruct((NB1, Q, G1), jnp.float32),
            jax.ShapeDtypeStruct((NTILES, Q, BT), jnp.float32),
            jax.ShapeDtypeStruct((NTILES, Q, KT // G2), jnp.float32),
        ],
    )(queries, keys_padded)


_GATHER_DNUMS = lax.GatherDimensionNumbers(
    offset_dims=(), collapsed_slice_dims=(0,), start_index_map=(0,))


def _permute(x, perm):
    return lax.gather(x, perm[:, None], _GATHER_DNUMS, slice_sizes=(1,),
                      mode=lax.GatherScatterMode.PROMISE_IN_BOUNDS)


def _merge16(av, ai, bv, bi):
    rbv = lax.rev(bv, (0,))
    rbi = lax.rev(bi, (0,))
    take = (av > rbv) | ((av == rbv) & (ai < rbi))
    cv = jnp.where(take, av, rbv)
    ci = jnp.where(take, ai, rbi)
    cv, ci = plsc.sort_key_val(cv, ci, descending=True)
    return cv, ci


def _merge_tree(pairs):
    while len(pairs) > 1:
        nxt = []
        for a in range(0, len(pairs) - 1, 2):
            (av, ai), (bv, bi) = pairs[a], pairs[a + 1]
            nxt.append(_merge16(av, ai, bv, bi))
        if len(pairs) % 2:
            nxt.append(pairs[-1])
        pairs = nxt
    return pairs[0]


def _tie_fix(v, i):
    lane = lax.iota(jnp.int32, L)
    for parity in (0, 1, 0, 1):
        partner = lane - 2 * ((lane - parity) % 2) + 1
        partner = jnp.clip(partner, 0, L - 1)
        pv = _permute(v, partner)
        pi = _permute(i, partner)
        win = (v > pv) | ((v == pv) & (i < pi)) | (partner == lane)
        first = partner > lane
        keep_self = jnp.where(first, win, ~win)
        v = jnp.where(keep_self, v, pv)
        i = jnp.where(keep_self, i, pi)
    return v, i


def _topk_sc_body(m2_hbm, m1_hbm, sc_hbm, vals_hbm, idx_hbm,
                  m2_v, m1_v, gidx_v, gath_v, ov_v, oi_v, sem):
    wid = lax.axis_index("c") * NS + lax.axis_index("s")
    base = wid * QW
    pltpu.sync_copy(m2_hbm.at[pl.ds(base * NB2P, QW * NB2P)], m2_v)
    pltpu.sync_copy(m1_hbm.at[pl.ds(base * NB1, QW * NB1)], m1_v)

    lane = lax.iota(jnp.int32, L)

    def one_query(r, _):
        q = base + r
        pairs = []
        for c in range(NB2P // L):
            v = m2_v[pl.ds(r * NB2P + c * L, L)]
            i = lane + c * L
            pairs.append(plsc.sort_key_val(v, i, descending=True))
        sbv, sbi = _merge_tree(pairs)

        pairs = []
        for g in range(G2 // G1):
            i = sbi * (G2 // G1) + g
            v = plsc.load_gather(m1_v, [r * NB1 + i])
            pairs.append(plsc.sort_key_val(v, i, descending=True))
        bv, bi = _merge_tree(pairs)

        gidx_v[...] = bi * Q + q
        pltpu.async_copy(sc_hbm.at[gidx_v], gath_v, sem).wait()

        pairs = []
        for blk in range(L):
            b = _permute(bi, jnp.full((L,), blk, jnp.int32))
            for seg in range(G1 // L):
                v = gath_v[blk, pl.ds(seg * L, L)]
                i = b * G1 + (seg * L) + lane
                pairs.append(plsc.sort_key_val(v, i, descending=True))
        fv, fi = _merge_tree(pairs)
        fv, fi = _tie_fix(fv, fi)

        ov_v[pl.ds(r * L, L)] = fv
        oi_v[pl.ds(r * L, L)] = fi
        return 0

    lax.fori_loop(0, QW, one_query, 0)
    pltpu.sync_copy(ov_v, vals_hbm.at[pl.ds(base * L, QW * L)])
    pltpu.sync_copy(oi_v, idx_hbm.at[pl.ds(base * L, QW * L)])


def _topk_sc(m2_flat, m1_flat, scores_rows):
    kern = pl.kernel(
        _topk_sc_body,
        out_type=[
            jax.ShapeDtypeStruct((Q * L,), jnp.float32),
            jax.ShapeDtypeStruct((Q * L,), jnp.int32),
        ],
        mesh=plsc.VectorSubcoreMesh(core_axis_name="c", subcore_axis_name="s"),
        compiler_params=pltpu.CompilerParams(needs_layout_passes=False),
        scratch_types=[
            pltpu.VMEM((QW * NB2P,), jnp.float32),
            pltpu.VMEM((QW * NB1,), jnp.float32),
            pltpu.VMEM((L,), jnp.int32),
            pltpu.VMEM((L, G1), jnp.float32),
            pltpu.VMEM((QW * L,), jnp.float32),
            pltpu.VMEM((QW * L,), jnp.int32),
            pltpu.SemaphoreType.DMA,
        ],
    )
    return kern(m2_flat, m1_flat, scores_rows)


def kernel(queries, keys, k):
    pad = jnp.zeros((KPAD - K, D), keys.dtype).at[:, 0].set(1e18)
    keys_padded = jnp.concatenate([keys, pad], axis=0)
    scores, m1_3d, m2_3d = _phase1(queries, keys_padded)
    m1 = m1_3d.transpose(1, 0, 2).reshape(Q, NB1)
    m2 = m2_3d.transpose(1, 0, 2).reshape(Q, NB2)
    m2 = jnp.pad(m2, ((0, 0), (0, NB2P - NB2)), constant_values=NEG)
    vals_flat, idx_flat = _topk_sc(
        m2.reshape(-1), m1.reshape(-1), scores.reshape(NB1 * Q, G1))
    qsq = jnp.sum(queries * queries, axis=1, keepdims=True)
    vals = vals_flat.reshape(Q, L) - qsq
    idx = idx_flat.reshape(Q, L)
    idx = idx + jnp.asarray(k - k, idx.dtype)
    return vals, idx


# speedup vs baseline: 18.0573x; 1.2711x over previous
import jax
import jax.numpy as jnp
from jax import lax
from jax.experimental import pallas as pl
from jax.experimental.pallas import tpu as pltpu
from jax.experimental.pallas import tpu_sc as plsc

Q = 1024
D = 128
K = 100000
KT = 2048
NTILES = 49
KPAD = NTILES * KT
G1 = 128
G2 = 1024
NB1 = KPAD // G1
NB1P = 896
NB2 = KPAD // G2
NB2P = 128
BT = KT // G1
NEG = -3.0e38

NC = 2
NS = 16
L = 16
NW = NC * NS
QW = Q // NW


def _phase1_body(q_ref, k_ref, t_ref, s_ref, m1_ref):
    j = pl.program_id(0)
    q2 = q_ref[...] * 2.0

    def work(kt_ref):
        rows = []
        for sub in range(KT // 256):
            kk = kt_ref[pl.ds(sub * 256, 256), :]
            dot2 = lax.dot_general(
                q2, kk, (((1,), (1,)), ((), ())),
                preferred_element_type=jnp.float32,
                precision=lax.Precision.DEFAULT,
            )
            ksq = jnp.sum(kk * kk, axis=1)[None, :]
            s = dot2 - ksq
            for h in range(2):
                lb = sub * 2 + h
                sh = s[:, h * G1:(h + 1) * G1]
                s_ref[lb] = sh
                rows.append(jnp.max(sh, axis=1, keepdims=True))
        m1t = jnp.concatenate(rows, axis=1)
        for c in range(8):
            @pl.when(j % 8 == c)
            def _(c=c):
                m1_ref[:, c * BT:(c + 1) * BT] = m1t

    @pl.when(j < NTILES - 1)
    def _():
        work(k_ref)

    @pl.when(j == NTILES - 1)
    def _():
        work(t_ref)


def _phase1(queries, keys, tail):
    return pl.pallas_call(
        _phase1_body,
        grid=(NTILES,),
        in_specs=[
            pl.BlockSpec((Q, D), lambda j: (0, 0)),
            pl.BlockSpec((KT, D), lambda j: (jnp.minimum(j, NTILES - 2), 0)),
            pl.BlockSpec((KT, D), lambda j: (0, 0)),
        ],
        out_specs=[
            pl.BlockSpec((BT, Q, G1), lambda j: (j, 0, 0)),
            pl.BlockSpec((Q, 128), lambda j: (0, j // 8)),
        ],
        out_shape=[
            jax.ShapeDtypeStruct((NB1, Q, G1), jnp.float32),
            jax.ShapeDtypeStruct((Q, NB1P), jnp.float32),
        ],
    )(queries, keys, tail)


_GATHER_DNUMS = lax.GatherDimensionNumbers(
    offset_dims=(), collapsed_slice_dims=(0,), start_index_map=(0,))


def _permute(x, perm):
    return lax.gather(x, perm[:, None], _GATHER_DNUMS, slice_sizes=(1,),
                      mode=lax.GatherScatterMode.PROMISE_IN_BOUNDS)


def _merge16(av, ai, bv, bi):
    rbv = lax.rev(bv, (0,))
    rbi = lax.rev(bi, (0,))
    take = (av > rbv) | ((av == rbv) & (ai < rbi))
    cv = jnp.where(take, av, rbv)
    ci = jnp.where(take, ai, rbi)
    cv, ci = plsc.sort_key_val(cv, ci, descending=True)
    return cv, ci


def _merge_tree(pairs):
    while len(pairs) > 1:
        nxt = []
        for a in range(0, len(pairs) - 1, 2):
            (av, ai), (bv, bi) = pairs[a], pairs[a + 1]
            nxt.append(_merge16(av, ai, bv, bi))
        if len(pairs) % 2:
            nxt.append(pairs[-1])
        pairs = nxt
    return pairs[0]


def _tie_fix(v, i):
    lane = lax.iota(jnp.int32, L)
    for parity in (0, 1, 0, 1):
        partner = lane - 2 * ((lane - parity) % 2) + 1
        partner = jnp.clip(partner, 0, L - 1)
        pv = _permute(v, partner)
        pi = _permute(i, partner)
        win = (v > pv) | ((v == pv) & (i < pi)) | (partner == lane)
        first = partner > lane
        keep_self = jnp.where(first, win, ~win)
        v = jnp.where(keep_self, v, pv)
        i = jnp.where(keep_self, i, pi)
    return v, i


def _topk_sc_body(m1_hbm, sc_hbm, vals_hbm, idx_hbm,
                  m1_v, gidx_v, gath_v, ov_v, oi_v, sem):
    wid = lax.axis_index("c") * NS + lax.axis_index("s")
    base = wid * QW
    pltpu.sync_copy(m1_hbm.at[pl.ds(base * NB1P, QW * NB1P)], m1_v)

    lane = lax.iota(jnp.int32, L)

    def one_query(r, _):
        q = base + r
        pairs = []
        for c in range(NB1 // L):
            v = m1_v[pl.ds(r * NB1P + c * L, L)]
            i = lane + c * L
            pairs.append(plsc.sort_key_val(v, i, descending=True))
        bv, bi = _merge_tree(pairs)

        gidx_v[...] = bi * Q + q
        pltpu.async_copy(sc_hbm.at[gidx_v], gath_v, sem).wait()

        pairs = []
        for blk in range(L):
            b = _permute(bi, jnp.full((L,), blk, jnp.int32))
            for seg in range(G1 // L):
                v = gath_v[blk, pl.ds(seg * L, L)]
                i = b * G1 + (seg * L) + lane
                pairs.append(plsc.sort_key_val(v, i, descending=True))
        fv, fi = _merge_tree(pairs)
        fv, fi = _tie_fix(fv, fi)

        ov_v[pl.ds(r * L, L)] = fv
        oi_v[pl.ds(r * L, L)] = fi
        return 0

    lax.fori_loop(0, QW, one_query, 0)
    pltpu.sync_copy(ov_v, vals_hbm.at[pl.ds(base * L, QW * L)])
    pltpu.sync_copy(oi_v, idx_hbm.at[pl.ds(base * L, QW * L)])


def _topk_sc(m1_flat, scores_rows):
    kern = pl.kernel(
        _topk_sc_body,
        out_type=[
            jax.ShapeDtypeStruct((Q * L,), jnp.float32),
            jax.ShapeDtypeStruct((Q * L,), jnp.int32),
        ],
        mesh=plsc.VectorSubcoreMesh(core_axis_name="c", subcore_axis_name="s"),
        compiler_params=pltpu.CompilerParams(needs_layout_passes=False),
        scratch_types=[
            pltpu.VMEM((QW * NB1P,), jnp.float32),
            pltpu.VMEM((L,), jnp.int32),
            pltpu.VMEM((L, G1), jnp.float32),
            pltpu.VMEM((QW * L,), jnp.float32),
            pltpu.VMEM((QW * L,), jnp.int32),
            pltpu.SemaphoreType.DMA,
        ],
    )
    return kern(m1_flat, scores_rows)


def kernel(queries, keys, k):
    pad = jnp.zeros((KPAD - K, D), keys.dtype).at[:, 0].set(1e18)
    tail = jnp.concatenate([keys[(NTILES - 1) * KT:], pad], axis=0)
    scores, m1q = _phase1(queries, keys, tail)
    vals_flat, idx_flat = _topk_sc(
        m1q.reshape(-1), scores.reshape(NB1 * Q, G1))
    qsq = jnp.sum(queries * queries, axis=1, keepdims=True)
    vals = vals_flat.reshape(Q, L) - qsq
    idx = idx_flat.reshape(Q, L)
    idx = idx + jnp.asarray(k - k, idx.dtype)
    return vals, idx


# speedup vs baseline: 18.1956x; 1.0077x over previous
import jax
import jax.numpy as jnp
from jax import lax
from jax.experimental import pallas as pl
from jax.experimental.pallas import tpu as pltpu
from jax.experimental.pallas import tpu_sc as plsc

Q = 1024
D = 128
K = 100000
KT = 4096
NTILES = 25
KPAD = NTILES * KT
G1 = 128
G2 = 1024
NB1 = KPAD // G1
NB1P = 896
NB2 = KPAD // G2
NB2P = 128
BT = KT // G1
NEG = -3.0e38

NC = 2
NS = 16
L = 16
NW = NC * NS
QW = Q // NW


def _phase1_body(q_ref, k_ref, t_ref, s_ref, m1_ref):
    j = pl.program_id(0)
    q2 = q_ref[...] * 2.0

    def work(kt_ref):
        rows = []
        for sub in range(KT // 256):
            kk = kt_ref[pl.ds(sub * 256, 256), :]
            dot2 = lax.dot_general(
                q2, kk, (((1,), (1,)), ((), ())),
                preferred_element_type=jnp.float32,
                precision=lax.Precision.DEFAULT,
            )
            ksq = jnp.sum(kk * kk, axis=1)[None, :]
            s = dot2 - ksq
            for h in range(2):
                lb = sub * 2 + h
                sh = s[:, h * G1:(h + 1) * G1]
                s_ref[lb] = sh
                rows.append(jnp.max(sh, axis=1, keepdims=True))
        m1t = jnp.concatenate(rows, axis=1)
        for c in range(128 // BT):
            @pl.when(j % (128 // BT) == c)
            def _(c=c):
                m1_ref[:, c * BT:(c + 1) * BT] = m1t

    @pl.when(j < NTILES - 1)
    def _():
        work(k_ref)

    @pl.when(j == NTILES - 1)
    def _():
        work(t_ref)


def _phase1(queries, keys, tail):
    return pl.pallas_call(
        _phase1_body,
        grid=(NTILES,),
        in_specs=[
            pl.BlockSpec((Q, D), lambda j: (0, 0)),
            pl.BlockSpec((KT, D), lambda j: (jnp.minimum(j, NTILES - 2), 0)),
            pl.BlockSpec((KT, D), lambda j: (0, 0)),
        ],
        out_specs=[
            pl.BlockSpec((BT, Q, G1), lambda j: (j, 0, 0)),
            pl.BlockSpec((Q, 128), lambda j: (0, j // (128 // BT))),
        ],
        out_shape=[
            jax.ShapeDtypeStruct((NB1, Q, G1), jnp.float32),
            jax.ShapeDtypeStruct((Q, NB1P), jnp.float32),
        ],
    )(queries, keys, tail)


_GATHER_DNUMS = lax.GatherDimensionNumbers(
    offset_dims=(), collapsed_slice_dims=(0,), start_index_map=(0,))


def _permute(x, perm):
    return lax.gather(x, perm[:, None], _GATHER_DNUMS, slice_sizes=(1,),
                      mode=lax.GatherScatterMode.PROMISE_IN_BOUNDS)


def _merge16(av, ai, bv, bi):
    rbv = lax.rev(bv, (0,))
    rbi = lax.rev(bi, (0,))
    take = (av > rbv) | ((av == rbv) & (ai < rbi))
    cv = jnp.where(take, av, rbv)
    ci = jnp.where(take, ai, rbi)
    cv, ci = plsc.sort_key_val(cv, ci, descending=True)
    return cv, ci


def _merge_tree(pairs):
    while len(pairs) > 1:
        nxt = []
        for a in range(0, len(pairs) - 1, 2):
            (av, ai), (bv, bi) = pairs[a], pairs[a + 1]
            nxt.append(_merge16(av, ai, bv, bi))
        if len(pairs) % 2:
            nxt.append(pairs[-1])
        pairs = nxt
    return pairs[0]


def _tie_fix(v, i):
    lane = lax.iota(jnp.int32, L)
    for parity in (0, 1, 0, 1):
        partner = lane - 2 * ((lane - parity) % 2) + 1
        partner = jnp.clip(partner, 0, L - 1)
        pv = _permute(v, partner)
        pi = _permute(i, partner)
        win = (v > pv) | ((v == pv) & (i < pi)) | (partner == lane)
        first = partner > lane
        keep_self = jnp.where(first, win, ~win)
        v = jnp.where(keep_self, v, pv)
        i = jnp.where(keep_self, i, pi)
    return v, i


def _topk_sc_body(m1_hbm, sc_hbm, vals_hbm, idx_hbm,
                  m1_v, gidx_v, gath_v, ov_v, oi_v, sem):
    wid = lax.axis_index("c") * NS + lax.axis_index("s")
    base = wid * QW
    pltpu.sync_copy(m1_hbm.at[pl.ds(base * NB1P, QW * NB1P)], m1_v)

    lane = lax.iota(jnp.int32, L)

    def one_query(r, _):
        q = base + r
        pairs = []
        for c in range(NB1 // L):
            v = m1_v[pl.ds(r * NB1P + c * L, L)]
            i = lane + c * L
            pairs.append(plsc.sort_key_val(v, i, descending=True))
        bv, bi = _merge_tree(pairs)

        gidx_v[...] = bi * Q + q
        pltpu.async_copy(sc_hbm.at[gidx_v], gath_v, sem).wait()

        pairs = []
        for blk in range(L):
            b = _permute(bi, jnp.full((L,), blk, jnp.int32))
            for seg in range(G1 // L):
                v = gath_v[blk, pl.ds(seg * L, L)]
                i = b * G1 + (seg * L) + lane
                pairs.append(plsc.sort_key_val(v, i, descending=True))
        fv, fi = _merge_tree(pairs)
        fv, fi = _tie_fix(fv, fi)

        ov_v[pl.ds(r * L, L)] = fv
        oi_v[pl.ds(r * L, L)] = fi
        return 0

    lax.fori_loop(0, QW, one_query, 0)
    pltpu.sync_copy(ov_v, vals_hbm.at[pl.ds(base * L, QW * L)])
    pltpu.sync_copy(oi_v, idx_hbm.at[pl.ds(base * L, QW * L)])


def _topk_sc(m1_flat, scores_rows):
    kern = pl.kernel(
        _topk_sc_body,
        out_type=[
            jax.ShapeDtypeStruct((Q * L,), jnp.float32),
            jax.ShapeDtypeStruct((Q * L,), jnp.int32),
        ],
        mesh=plsc.VectorSubcoreMesh(core_axis_name="c", subcore_axis_name="s"),
        compiler_params=pltpu.CompilerParams(needs_layout_passes=False),
        scratch_types=[
            pltpu.VMEM((QW * NB1P,), jnp.float32),
            pltpu.VMEM((L,), jnp.int32),
            pltpu.VMEM((L, G1), jnp.float32),
            pltpu.VMEM((QW * L,), jnp.float32),
            pltpu.VMEM((QW * L,), jnp.int32),
            pltpu.SemaphoreType.DMA,
        ],
    )
    return kern(m1_flat, scores_rows)


def kernel(queries, keys, k):
    pad = jnp.zeros((KPAD - K, D), keys.dtype).at[:, 0].set(1e18)
    tail = jnp.concatenate([keys[(NTILES - 1) * KT:], pad], axis=0)
    scores, m1q = _phase1(queries, keys, tail)
    vals_flat, idx_flat = _topk_sc(
        m1q.reshape(-1), scores.reshape(NB1 * Q, G1))
    qsq = jnp.sum(queries * queries, axis=1, keepdims=True)
    vals = vals_flat.reshape(Q, L) - qsq
    idx = idx_flat.reshape(Q, L)
    idx = idx + jnp.asarray(k - k, idx.dtype)
    return vals, idx
